# Initial kernel scaffold; baseline (speedup 1.0000x reference)
import jax
import jax.numpy as jnp
from jax.experimental import pallas as pl


def kernel(x, edge_index, W1, b1, W2, b2, W3, b3):
    raise NotImplementedError("write your pallas kernel here")


# speedup vs baseline: 21.4378x; 21.4378x over previous
import functools

import jax
import jax.numpy as jnp
from jax import lax
from jax.experimental import pallas as pl
from jax.experimental.pallas import tpu as pltpu
from jax.experimental.pallas import tpu_sc as plsc

N_SUBCORES = 16
N_TILES = N_SUBCORES
CHUNK = 128
SUPER = 8
F = 16


def _pad_rows(n):
    rpt = -(-(n + 1) // N_SUBCORES)
    rpt = -(-rpt // 8) * 8
    return rpt * N_SUBCORES, rpt


_SC_PARAMS = pltpu.CompilerParams(use_tc_tiling_on_sc=False)


@functools.lru_cache(maxsize=None)
def _sc_aggregate(n_pad, nch):
    rpt = n_pad // N_SUBCORES
    n_super = nch // SUPER
    mesh = plsc.VectorSubcoreMesh(core_axis_name="c", subcore_axis_name="s",
                                  num_cores=1)

    @functools.partial(
        pl.kernel,
        mesh=mesh,
        out_type=jax.ShapeDtypeStruct((n_pad, F), jnp.float32),
        scratch_types=[
            pltpu.VMEM_SHARED((n_pad, F), jnp.float32),
            pltpu.VMEM((SUPER, CHUNK), jnp.int32),
            pltpu.VMEM((SUPER, CHUNK), jnp.int32),
            pltpu.VMEM((SUPER, CHUNK, F), jnp.float32),
            pltpu.SemaphoreType.DMA,
        ],
        compiler_params=_SC_PARAMS,
    )
    def k(table, src2, dst2, zeros, out, acc, src_v, dst_v, rows_v, sem):
        sid = lax.axis_index("s")

        pltpu.sync_copy(zeros.at[pl.ds(sid * rpt, rpt)],
                        acc.at[pl.ds(sid * rpt, rpt)])
        plsc.subcore_barrier()

        def body(s, _):
            base = sid * nch + s * SUPER
            pltpu.sync_copy(src2.at[pl.ds(base, SUPER)], src_v)
            pltpu.sync_copy(dst2.at[pl.ds(base, SUPER)], dst_v)
            gathers = []
            for j in range(SUPER):
                gathers.append(
                    pltpu.async_copy(table.at[src_v.at[j]], rows_v.at[j], sem))
            for g in gathers:
                g.wait()
            scatters = []
            for j in range(SUPER):
                scatters.append(
                    pltpu.async_copy(rows_v.at[j], acc.at[dst_v.at[j]], sem,
                                     add=True))
            for sc in scatters:
                sc.wait()
            return 0
        lax.fori_loop(0, n_super, body, 0)
        plsc.subcore_barrier()

        pltpu.sync_copy(acc.at[pl.ds(sid * rpt, rpt)],
                        out.at[pl.ds(sid * rpt, rpt)])

    return k


_ROWS = 3128


def _row_spec():
    return pl.BlockSpec((_ROWS, F), lambda i: (i, 0))


def _w_spec(shape):
    return pl.BlockSpec(shape, lambda i: (0, 0))


@functools.lru_cache(maxsize=None)
def _tc_pre(n_pad):
    def body(s_deg, xp, dis_o, g1_o):
        deg = s_deg[:, 0:1] + 1.0
        dis = lax.rsqrt(deg)
        dis16 = jnp.broadcast_to(dis, dis_o.shape)
        dis_o[...] = dis16
        g1_o[...] = dis16 * xp[...]

    grid = (n_pad // _ROWS,)
    return pl.pallas_call(
        body,
        grid=grid,
        in_specs=[_row_spec(), _row_spec()],
        out_specs=[_row_spec(), _row_spec()],
        out_shape=[jax.ShapeDtypeStruct((n_pad, F), jnp.float32)] * 2,
    )


@functools.lru_cache(maxsize=None)
def _tc_layer1(n_pad):
    def body(s1, g1, dis, w1p, b1, g2a_o, g2b_o):
        u1 = dis[...] * (s1[...] + g1[...])
        h1 = jnp.maximum(
            jnp.dot(u1, w1p[...], preferred_element_type=jnp.float32)
            + b1[...], 0.0)
        g2a_o[...] = dis[...] * h1[:, :F]
        g2b_o[...] = dis[...] * h1[:, F:]

    grid = (n_pad // _ROWS,)
    return pl.pallas_call(
        body,
        grid=grid,
        in_specs=[_row_spec(), _row_spec(), _row_spec(),
                  _w_spec((F, 32)), _w_spec((1, 32))],
        out_specs=[_row_spec(), _row_spec()],
        out_shape=[jax.ShapeDtypeStruct((n_pad, F), jnp.float32)] * 2,
    )


@functools.lru_cache(maxsize=None)
def _tc_layer2(n_pad):
    def body(s2a, g2a, s2b, g2b, dis, w2a, w2b, w3p, b2, g3_o):
        u2a = dis[...] * (s2a[...] + g2a[...])
        u2b = dis[...] * (s2b[...] + g2b[...])
        h2 = jnp.maximum(
            jnp.dot(u2a, w2a[...], preferred_element_type=jnp.float32)
            + jnp.dot(u2b, w2b[...], preferred_element_type=jnp.float32)
            + b2[...], 0.0)
        g3_o[...] = dis[...] * jnp.dot(h2, w3p[...],
                                       preferred_element_type=jnp.float32)

    grid = (n_pad // _ROWS,)
    return pl.pallas_call(
        body,
        grid=grid,
        in_specs=[_row_spec()] * 5 + [_w_spec((F, 32)), _w_spec((F, 32)),
                                      _w_spec((32, F)), _w_spec((1, 32))],
        out_specs=_row_spec(),
        out_shape=jax.ShapeDtypeStruct((n_pad, F), jnp.float32),
    )


@functools.lru_cache(maxsize=None)
def _tc_layer3(n_pad):
    def body(s3, g3, dis, b3, out_o):
        out_o[...] = dis[...] * (s3[...] + g3[...]) + b3[...]

    grid = (n_pad // _ROWS,)
    return pl.pallas_call(
        body,
        grid=grid,
        in_specs=[_row_spec()] * 3 + [_w_spec((1, F))],
        out_specs=_row_spec(),
        out_shape=jax.ShapeDtypeStruct((n_pad, F), jnp.float32),
    )


def kernel(x, edge_index, W1, b1, W2, b2, W3, b3):
    n = x.shape[0]
    e = edge_index.shape[1]
    n_pad, _ = _pad_rows(n)
    dump = n
    epc = N_TILES * CHUNK * SUPER
    nch = (-(-e // epc)) * SUPER
    e_pad = nch * N_TILES * CHUNK

    ei = edge_index.astype(jnp.int32)
    pad = jnp.full((e_pad - e,), dump, jnp.int32)
    src2 = jnp.concatenate([ei[0], pad]).reshape(N_TILES * nch, CHUNK)
    dst2 = jnp.concatenate([ei[1], pad]).reshape(N_TILES * nch, CHUNK)

    xp = jnp.pad(x, ((0, n_pad - n), (0, F - x.shape[1])))
    w1p = jnp.pad(W1, ((0, F - W1.shape[0]), (0, 0)))
    w2a, w2b = W2[:F], W2[F:]
    w3p = jnp.pad(W3, ((0, 0), (0, F - W3.shape[1])))
    b1r = b1.reshape(1, 32)
    b2r = b2.reshape(1, 32)
    b3r = jnp.broadcast_to(b3.reshape(1, 1), (1, F))

    zeros16 = jnp.zeros((n_pad, F), jnp.float32)
    e0_table = jnp.tile(
        jnp.eye(1, F, dtype=jnp.float32), (n_pad, 1))

    agg = _sc_aggregate(n_pad, nch)

    def br_pre(s, st):
        dis16, g1 = _tc_pre(n_pad)(s, xp)
        return {**st, "table": g1, "dis": dis16, "g1": g1}

    def br_l1(s, st):
        g2a, g2b = _tc_layer1(n_pad)(s, st["g1"], st["dis"], w1p, b1r)
        return {**st, "table": g2a, "g2a": g2a, "g2b": g2b}

    def br_stash(s, st):
        return {**st, "table": st["g2b"], "s2a": s}

    def br_l2(s, st):
        g3 = _tc_layer2(n_pad)(st["s2a"], st["g2a"], s, st["g2b"],
                               st["dis"], w2a, w2b, w3p, b2r)
        return {**st, "table": g3, "g3": g3}

    def br_l3(s, st):
        out16 = _tc_layer3(n_pad)(s, st["g3"], st["dis"], b3r)
        return {**st, "out": out16}

    state = {
        "table": e0_table, "dis": zeros16, "g1": zeros16, "g2a": zeros16,
        "g2b": zeros16, "s2a": zeros16, "g3": zeros16, "out": zeros16,
    }

    def step(i, st):
        s = agg(st["table"], src2, dst2, zeros16)
        return lax.switch(i, [br_pre, br_l1, br_stash, br_l2, br_l3], s, st)

    state = lax.fori_loop(0, 5, step, state)
    return state["out"][:n, 0]


# speedup vs baseline: 24.3293x; 1.1349x over previous
import functools

import jax
import jax.numpy as jnp
from jax import lax
from jax.experimental import pallas as pl
from jax.experimental.pallas import tpu as pltpu
from jax.experimental.pallas import tpu_sc as plsc

N_SUBCORES = 16
N_TILES = N_SUBCORES
CHUNK = 128
SUPER = 8
F = 16


def _pad_rows(n):
    rpt = -(-(n + 1) // N_SUBCORES)
    rpt = -(-rpt // 8) * 8
    return rpt * N_SUBCORES, rpt


_SC_PARAMS = pltpu.CompilerParams(use_tc_tiling_on_sc=False,
                                  needs_layout_passes=False)


HALF = SUPER // 2


@functools.lru_cache(maxsize=None)
def _sc_aggregate(n_pad, nch):
    rpt = n_pad // N_SUBCORES
    n_super = nch // SUPER
    mesh = plsc.VectorSubcoreMesh(core_axis_name="c", subcore_axis_name="s",
                                  num_cores=1)

    @functools.partial(
        pl.kernel,
        mesh=mesh,
        out_type=jax.ShapeDtypeStruct((n_pad, F), jnp.float32),
        scratch_types=[
            pltpu.VMEM_SHARED((n_pad, F), jnp.float32),
            pltpu.VMEM((SUPER, CHUNK), jnp.int32),
            pltpu.VMEM((SUPER, CHUNK), jnp.int32),
            pltpu.VMEM((HALF, CHUNK, F), jnp.float32),
            pltpu.VMEM((HALF, CHUNK, F), jnp.float32),
            pltpu.VMEM((F,), jnp.int32),
            pltpu.SemaphoreType.DMA,
            pltpu.SemaphoreType.DMA,
        ],
        compiler_params=_SC_PARAMS,
    )
    def k(table, src2, dst2, zeros, mode, out,
          acc, src_v, dst_v, rows_a, rows_b, mode_v, gsem, ssem):
        sid = lax.axis_index("s")

        pltpu.sync_copy(mode, mode_v)
        is_gather = jnp.max(mode_v[...]) == 1

        pltpu.sync_copy(zeros.at[pl.ds(sid * rpt, rpt)],
                        acc.at[pl.ds(sid * rpt, rpt)])

        @pl.when(jnp.logical_not(is_gather))
        def _():
            one0 = jnp.where(lax.iota(jnp.int32, F) == 0,
                             jnp.float32(1.0), jnp.float32(0.0))

            def fill(i, _):
                rows_a[i // CHUNK, i % CHUNK, :] = one0
                rows_b[i // CHUNK, i % CHUNK, :] = one0
                return 0
            lax.fori_loop(0, HALF * CHUNK, fill, 0)

        plsc.subcore_barrier()

        def scatter_half(rows, dst_idx):
            return [pltpu.async_copy(rows.at[j], acc.at[dst_idx.at[j]],
                                     ssem, add=True)
                    for j in range(HALF)]

        def body(s, _):
            base = sid * nch + s * SUPER
            pltpu.sync_copy(src2.at[pl.ds(base, SUPER)], src_v)
            pltpu.sync_copy(dst2.at[pl.ds(base, SUPER)], dst_v)

            @pl.when(is_gather)
            def _():
                g = [pltpu.async_copy(table.at[src_v.at[j]],
                                      (rows_a if j < HALF else rows_b)
                                      .at[j % HALF], gsem)
                     for j in range(SUPER)]
                for d in g[:HALF]:
                    d.wait()
                sa = scatter_half(rows_a, dst_v)
                for d in g[HALF:]:
                    d.wait()
                sb = [pltpu.async_copy(rows_b.at[j],
                                      acc.at[dst_v.at[HALF + j]], ssem,
                                      add=True)
                      for j in range(HALF)]
                for d in sa + sb:
                    d.wait()

            @pl.when(jnp.logical_not(is_gather))
            def _():
                sa = scatter_half(rows_a, dst_v)
                sb = [pltpu.async_copy(rows_b.at[j],
                                      acc.at[dst_v.at[HALF + j]], ssem,
                                      add=True)
                      for j in range(HALF)]
                for d in sa + sb:
                    d.wait()
            return 0
        lax.fori_loop(0, n_super, body, 0)
        plsc.subcore_barrier()

        pltpu.sync_copy(acc.at[pl.ds(sid * rpt, rpt)],
                        out.at[pl.ds(sid * rpt, rpt)])

    return k


_ROWS = 3128


def _row_spec():
    return pl.BlockSpec((_ROWS, F), lambda i: (i, 0))


def _w_spec(shape):
    return pl.BlockSpec(shape, lambda i: (0, 0))


@functools.lru_cache(maxsize=None)
def _tc_pre(n_pad):
    def body(s_deg, xp, dis_o, g1_o):
        deg = s_deg[:, 0:1] + 1.0
        dis = lax.rsqrt(deg)
        dis16 = jnp.broadcast_to(dis, dis_o.shape)
        dis_o[...] = dis16
        g1_o[...] = dis16 * xp[...]

    grid = (n_pad // _ROWS,)
    return pl.pallas_call(
        body,
        grid=grid,
        in_specs=[_row_spec(), _row_spec()],
        out_specs=[_row_spec(), _row_spec()],
        out_shape=[jax.ShapeDtypeStruct((n_pad, F), jnp.float32)] * 2,
    )


@functools.lru_cache(maxsize=None)
def _tc_layer1(n_pad):
    def body(s1, g1, dis, w1p, b1, g2a_o, g2b_o):
        u1 = dis[...] * (s1[...] + g1[...])
        h1 = jnp.maximum(
            jnp.dot(u1, w1p[...], preferred_element_type=jnp.float32)
            + b1[...], 0.0)
        g2a_o[...] = dis[...] * h1[:, :F]
        g2b_o[...] = dis[...] * h1[:, F:]

    grid = (n_pad // _ROWS,)
    return pl.pallas_call(
        body,
        grid=grid,
        in_specs=[_row_spec(), _row_spec(), _row_spec(),
                  _w_spec((F, 32)), _w_spec((1, 32))],
        out_specs=[_row_spec(), _row_spec()],
        out_shape=[jax.ShapeDtypeStruct((n_pad, F), jnp.float32)] * 2,
    )


@functools.lru_cache(maxsize=None)
def _tc_layer2(n_pad):
    def body(s2a, g2a, s2b, g2b, dis, w2a, w2b, w3p, b2, g3_o):
        u2a = dis[...] * (s2a[...] + g2a[...])
        u2b = dis[...] * (s2b[...] + g2b[...])
        h2 = jnp.maximum(
            jnp.dot(u2a, w2a[...], preferred_element_type=jnp.float32)
            + jnp.dot(u2b, w2b[...], preferred_element_type=jnp.float32)
            + b2[...], 0.0)
        g3_o[...] = dis[...] * jnp.dot(h2, w3p[...],
                                       preferred_element_type=jnp.float32)

    grid = (n_pad // _ROWS,)
    return pl.pallas_call(
        body,
        grid=grid,
        in_specs=[_row_spec()] * 5 + [_w_spec((F, 32)), _w_spec((F, 32)),
                                      _w_spec((32, F)), _w_spec((1, 32))],
        out_specs=_row_spec(),
        out_shape=jax.ShapeDtypeStruct((n_pad, F), jnp.float32),
    )


@functools.lru_cache(maxsize=None)
def _tc_layer3(n_pad):
    def body(s3, g3, dis, b3, out_o):
        out_o[...] = dis[...] * (s3[...] + g3[...]) + b3[...]

    grid = (n_pad // _ROWS,)
    return pl.pallas_call(
        body,
        grid=grid,
        in_specs=[_row_spec()] * 3 + [_w_spec((1, F))],
        out_specs=_row_spec(),
        out_shape=jax.ShapeDtypeStruct((n_pad, F), jnp.float32),
    )


def kernel(x, edge_index, W1, b1, W2, b2, W3, b3):
    n = x.shape[0]
    e = edge_index.shape[1]
    n_pad, _ = _pad_rows(n)
    dump = n
    epc = N_TILES * CHUNK * SUPER
    nch = (-(-e // epc)) * SUPER
    e_pad = nch * N_TILES * CHUNK

    ei = edge_index.astype(jnp.int32)
    pad = jnp.full((e_pad - e,), dump, jnp.int32)
    src2 = jnp.concatenate([ei[0], pad]).reshape(N_TILES * nch, CHUNK)
    dst2 = jnp.concatenate([ei[1], pad]).reshape(N_TILES * nch, CHUNK)

    xp = jnp.pad(x, ((0, n_pad - n), (0, F - x.shape[1])))
    w1p = jnp.pad(W1, ((0, F - W1.shape[0]), (0, 0)))
    w2a, w2b = W2[:F], W2[F:]
    w3p = jnp.pad(W3, ((0, 0), (0, F - W3.shape[1])))
    b1r = b1.reshape(1, 32)
    b2r = b2.reshape(1, 32)
    b3r = jnp.broadcast_to(b3.reshape(1, 1), (1, F))

    zeros16 = jnp.zeros((n_pad, F), jnp.float32)

    agg = _sc_aggregate(n_pad, nch)

    def br_pre(s, st):
        dis16, g1 = _tc_pre(n_pad)(s, xp)
        return {**st, "table": g1, "dis": dis16, "g1": g1}

    def br_l1(s, st):
        g2a, g2b = _tc_layer1(n_pad)(s, st["g1"], st["dis"], w1p, b1r)
        return {**st, "table": g2a, "g2a": g2a, "g2b": g2b}

    def br_stash(s, st):
        return {**st, "table": st["g2b"], "s2a": s}

    def br_l2(s, st):
        g3 = _tc_layer2(n_pad)(st["s2a"], st["g2a"], s, st["g2b"],
                               st["dis"], w2a, w2b, w3p, b2r)
        return {**st, "table": g3, "g3": g3}

    def br_l3(s, st):
        out16 = _tc_layer3(n_pad)(s, st["g3"], st["dis"], b3r)
        return {**st, "out": out16}

    state = {
        "table": zeros16, "dis": zeros16, "g1": zeros16, "g2a": zeros16,
        "g2b": zeros16, "s2a": zeros16, "g3": zeros16, "out": zeros16,
    }

    def step(i, st):
        mode = jnp.broadcast_to(jnp.where(i == 0, 0, 1).astype(jnp.int32),
                                (F,))
        s = agg(st["table"], src2, dst2, zeros16, mode)
        return lax.switch(i, [br_pre, br_l1, br_stash, br_l2, br_l3], s, st)

    state = lax.fori_loop(0, 5, step, state)
    return state["out"][:n, 0]
